# baseline (speedup 1.0000x reference)
import functools
import jax
import jax.numpy as jnp
from jax.experimental import pallas as pl
from jax.experimental.pallas import tpu as pltpu

ENC = 512
NODES = 63
BATCH = 16
LEVELS = [
    [62, 46, 38, 54, 34, 42, 50, 58, 32, 36, 40, 44, 48, 52, 56, 60],
    [30, 22, 18, 26, 16, 20, 24, 28],
    [14, 10, 8, 12],
    [6, 4],
    [2],
    [0],
]
ALL_NODES = [nd for level in LEVELS for nd in level]

_DN_T = (((1,), (1,)), ((), ()))


def _body(tok_ref, emb_hbm, wih_hbm, whh_hbm, bih_ref, bhh_ref, sw_hbm,
          sb_ref, cw_ref, out_ref, x_vmem, wih_vmem, whh_vmem, sw_vmem,
          sem_rows, sem_wih, sem_whh, sem_sw):
    B = BATCH
    wih_copy = pltpu.make_async_copy(wih_hbm, wih_vmem, sem_wih)
    wih_copy.start()

    chunk_copies = [[], []]
    r = 0
    for s, level in enumerate(LEVELS):
        chunk = 0 if s == 0 else 1
        for node in level:
            for b in range(B):
                tok = tok_ref[b, node]
                cp = pltpu.make_async_copy(emb_hbm.at[pl.ds(tok, 1)],
                                           x_vmem.at[pl.ds(r, 1)],
                                           sem_rows.at[chunk])
                cp.start()
                chunk_copies[chunk].append(cp)
                r += 1

    whh_copy = pltpu.make_async_copy(whh_hbm, whh_vmem, sem_whh)
    sw_copy = pltpu.make_async_copy(sw_hbm, sw_vmem, sem_sw)
    whh_copy.start()
    sw_copy.start()
    wih_copy.wait()

    b_ih = jnp.reshape(bih_ref[...], (1, 3 * ENC))
    b_hh = jnp.reshape(bhh_ref[...], (1, 3 * ENC))
    sb = sb_ref[...]
    cw = cw_ref[...]
    c0 = jnp.dot(jnp.tanh(sb), cw, preferred_element_type=jnp.float32)

    out = jnp.zeros((B, ENC), dtype=jnp.float32)
    half = 16 * B
    for cp in chunk_copies[0]:
        cp.wait()
    gi_a = jax.lax.dot_general(x_vmem[0:half], wih_vmem[...], _DN_T,
                               preferred_element_type=jnp.float32) + b_ih
    gi_b = None
    off = 0
    h_prev = None
    sw = None
    for s, level in enumerate(LEVELS):
        n = len(level) * B
        if s == 1:
            for cp in chunk_copies[1]:
                cp.wait()
            gi_b = jax.lax.dot_general(x_vmem[half:2 * half], wih_vmem[...],
                                       _DN_T,
                                       preferred_element_type=jnp.float32) + b_ih
        gi = gi_a if s == 0 else gi_b[off - half:off - half + n]
        if s == 0:
            c = jnp.zeros((n, ENC), dtype=jnp.float32)
            gh = jnp.broadcast_to(b_hh, (n, 3 * ENC))
        else:
            if s == 1:
                sw_copy.wait()
                whh_copy.wait()
                sw = sw_vmem[...]
            h_child = h_prev[:n]
            t = jnp.tanh(jnp.dot(h_child, sw,
                                 preferred_element_type=jnp.float32) + sb)
            l = jnp.tanh(jnp.dot(t, cw, preferred_element_type=jnp.float32))
            k = 15.0 if s == len(LEVELS) - 1 else 1.0
            gate = 1.0 / (1.0 + k * jnp.exp(c0 - l))
            c = h_child * gate
            gh = jax.lax.dot_general(c, whh_vmem[...], _DN_T,
                                     preferred_element_type=jnp.float32) + b_hh
        i_r = gi[:, 0:ENC]
        i_z = gi[:, ENC:2 * ENC]
        i_n = gi[:, 2 * ENC:3 * ENC]
        h_r = gh[:, 0:ENC]
        h_z = gh[:, ENC:2 * ENC]
        h_n = gh[:, 2 * ENC:3 * ENC]
        rr = jax.nn.sigmoid(i_r + h_r)
        z = jax.nn.sigmoid(i_z + h_z)
        nn_ = jnp.tanh(i_n + rr * h_n)
        h = (1.0 - z) * nn_ + z * c
        for i in range(len(level)):
            out = jnp.maximum(out, h[i * B:(i + 1) * B])
        h_prev = h
        off += n
    out_ref[...] = jnp.maximum(out, 0.0)


@jax.jit
def _run(tokens, emb, W_ih, W_hh, b_ih, b_hh, sent_weight, sent_bias,
         context_weight):
    vm = pltpu.MemorySpace.VMEM
    hbm = pltpu.MemorySpace.HBM
    grid_spec = pltpu.PrefetchScalarGridSpec(
        num_scalar_prefetch=1,
        grid=(1,),
        in_specs=[
            pl.BlockSpec(memory_space=hbm),
            pl.BlockSpec(memory_space=hbm),
            pl.BlockSpec(memory_space=hbm),
            pl.BlockSpec(memory_space=vm),
            pl.BlockSpec(memory_space=vm),
            pl.BlockSpec(memory_space=hbm),
            pl.BlockSpec(memory_space=vm),
            pl.BlockSpec(memory_space=vm),
        ],
        out_specs=pl.BlockSpec(memory_space=vm),
        scratch_shapes=[
            pltpu.VMEM((32 * BATCH, ENC), jnp.float32),
            pltpu.VMEM((3 * ENC, ENC), jnp.float32),
            pltpu.VMEM((3 * ENC, ENC), jnp.float32),
            pltpu.VMEM((ENC, ENC), jnp.float32),
            pltpu.SemaphoreType.DMA((2,)),
            pltpu.SemaphoreType.DMA,
            pltpu.SemaphoreType.DMA,
            pltpu.SemaphoreType.DMA,
        ],
    )
    out = pl.pallas_call(
        _body,
        grid_spec=grid_spec,
        out_shape=jax.ShapeDtypeStruct((BATCH, ENC), jnp.float32),
    )(tokens, emb, W_ih, W_hh, b_ih, b_hh, sent_weight, sent_bias,
      context_weight)
    return out


def kernel(tokens, bs, emb, W_ih, W_hh, b_ih, b_hh, sent_weight, sent_bias,
           context_weight):
    del bs
    return _run(tokens, emb, W_ih, W_hh, b_ih, b_hh, sent_weight,
                sent_bias, context_weight)


# speedup vs baseline: 1.1104x; 1.1104x over previous
import functools
import jax
import jax.numpy as jnp
from jax.experimental import pallas as pl
from jax.experimental.pallas import tpu as pltpu

ENC = 512
NODES = 63
BATCH = 16
LEVELS = [
    [62, 46, 38, 54, 34, 42, 50, 58, 32, 36, 40, 44, 48, 52, 56, 60],
    [30, 22, 18, 26, 16, 20, 24, 28],
    [14, 10, 8, 12],
    [6, 4],
    [2],
    [0],
]
ALL_NODES = [nd for level in LEVELS for nd in level]

_DN_T = (((1,), (1,)), ((), ()))


def _body(tok_ref, emb_hbm, wih_hbm, whh_hbm, bih_ref, bhh_ref, sw_hbm,
          sb_ref, cw_ref, out_ref, x_vmem, wih_vmem, whh_vmem, sw_vmem,
          sem_rows, sem_wih, sem_whh, sem_sw):
    B = BATCH
    wih_copy = pltpu.make_async_copy(wih_hbm, wih_vmem, sem_wih)
    wih_copy.start()

    chunk_copies = [[], []]
    r = 0
    for s, level in enumerate(LEVELS):
        chunk = 0 if s == 0 else 1
        for node in level:
            for b in range(B):
                tok = tok_ref[b, node]
                cp = pltpu.make_async_copy(emb_hbm.at[pl.ds(tok, 1)],
                                           x_vmem.at[pl.ds(r, 1)],
                                           sem_rows.at[chunk])
                cp.start()
                chunk_copies[chunk].append(cp)
                r += 1

    whh_copy = pltpu.make_async_copy(whh_hbm, whh_vmem, sem_whh)
    sw_copy = pltpu.make_async_copy(sw_hbm, sw_vmem, sem_sw)
    whh_copy.start()
    sw_copy.start()
    wih_copy.wait()

    b_ih = jnp.reshape(bih_ref[...], (1, 3 * ENC))
    b_hh = jnp.reshape(bhh_ref[...], (1, 3 * ENC))
    sb = sb_ref[...]
    cw = cw_ref[...]
    c0 = jnp.dot(jnp.tanh(sb), cw, preferred_element_type=jnp.float32)

    out = jnp.zeros((B, ENC), dtype=jnp.float32)
    half = 16 * B
    for cp in chunk_copies[0]:
        cp.wait()
    gi_a = jax.lax.dot_general(x_vmem[0:half], wih_vmem[...], _DN_T,
                               preferred_element_type=jnp.float32) + b_ih
    gi_b = None
    off = 0
    h_prev = None
    sw = None
    for s, level in enumerate(LEVELS):
        n = len(level) * B
        if s == 1:
            for cp in chunk_copies[1]:
                cp.wait()
            gi_b = jax.lax.dot_general(x_vmem[half:2 * half], wih_vmem[...],
                                       _DN_T,
                                       preferred_element_type=jnp.float32) + b_ih
        gi = gi_a if s == 0 else gi_b[off - half:off - half + n]
        if s == 0:
            c = jnp.zeros((n, ENC), dtype=jnp.float32)
            gh = jnp.broadcast_to(b_hh, (n, 3 * ENC))
        else:
            if s == 1:
                sw_copy.wait()
                whh_copy.wait()
                sw = sw_vmem[...]
            h_child = h_prev[:n]
            t = jnp.tanh(jnp.dot(h_child, sw,
                                 preferred_element_type=jnp.float32) + sb)
            hW = jax.lax.dot_general(h_child, whh_vmem[...], _DN_T,
                                     preferred_element_type=jnp.float32)
            l = jnp.tanh(jnp.dot(t, cw, preferred_element_type=jnp.float32))
            k = 15.0 if s == len(LEVELS) - 1 else 1.0
            gate = 1.0 / (1.0 + k * jnp.exp(c0 - l))
            c = h_child * gate
            gh = gate * hW + b_hh
        i_r = gi[:, 0:ENC]
        i_z = gi[:, ENC:2 * ENC]
        i_n = gi[:, 2 * ENC:3 * ENC]
        h_r = gh[:, 0:ENC]
        h_z = gh[:, ENC:2 * ENC]
        h_n = gh[:, 2 * ENC:3 * ENC]
        rr = jax.nn.sigmoid(i_r + h_r)
        z = jax.nn.sigmoid(i_z + h_z)
        nn_ = jnp.tanh(i_n + rr * h_n)
        h = (1.0 - z) * nn_ + z * c
        for i in range(len(level)):
            out = jnp.maximum(out, h[i * B:(i + 1) * B])
        h_prev = h
        off += n
    out_ref[...] = jnp.maximum(out, 0.0)


@jax.jit
def _run(tokens, emb, W_ih, W_hh, b_ih, b_hh, sent_weight, sent_bias,
         context_weight):
    vm = pltpu.MemorySpace.VMEM
    hbm = pltpu.MemorySpace.HBM
    smem = pltpu.MemorySpace.SMEM
    out = pl.pallas_call(
        _body,
        in_specs=[
            pl.BlockSpec(memory_space=smem),
            pl.BlockSpec(memory_space=hbm),
            pl.BlockSpec(memory_space=hbm),
            pl.BlockSpec(memory_space=hbm),
            pl.BlockSpec(memory_space=vm),
            pl.BlockSpec(memory_space=vm),
            pl.BlockSpec(memory_space=hbm),
            pl.BlockSpec(memory_space=vm),
            pl.BlockSpec(memory_space=vm),
        ],
        scratch_shapes=[
            pltpu.VMEM((32 * BATCH, ENC), jnp.float32),
            pltpu.VMEM((3 * ENC, ENC), jnp.float32),
            pltpu.VMEM((3 * ENC, ENC), jnp.float32),
            pltpu.VMEM((ENC, ENC), jnp.float32),
            pltpu.SemaphoreType.DMA((2,)),
            pltpu.SemaphoreType.DMA,
            pltpu.SemaphoreType.DMA,
            pltpu.SemaphoreType.DMA,
        ],
        out_shape=jax.ShapeDtypeStruct((BATCH, ENC), jnp.float32),
    )(tokens, emb, W_ih, W_hh, b_ih, b_hh, sent_weight, sent_bias,
      context_weight)
    return out


def kernel(tokens, bs, emb, W_ih, W_hh, b_ih, b_hh, sent_weight, sent_bias,
           context_weight):
    del bs
    return _run(tokens, emb, W_ih, W_hh, b_ih, b_hh, sent_weight,
                sent_bias, context_weight)
